# baseline (speedup 1.0000x reference)
import jax
import jax.numpy as jnp
from jax.experimental import pallas as pl
from jax.experimental.pallas import tpu as pltpu

B, L, D, F = 128, 2, 768, 32768
BF = 1024
NF = F // BF
NBUF = 4


def _we_copy(we_hbm, we_buf, we_sem, j, slot):
    return pltpu.make_async_copy(
        we_hbm.at[:, pl.ds(j * BF, BF)], we_buf.at[slot], we_sem.at[slot])


def _wd_copy(wd_hbm, wd_buf, wd_sem, j, slot):
    return pltpu.make_async_copy(
        wd_hbm.at[:, pl.ds(j * BF, BF), :], wd_buf.at[slot], wd_sem.at[slot])


def _body(x_ref, be_ref, bd_ref, we_hbm, wd_hbm, out0_ref, out1_ref,
          we_buf, wd_buf, we_sem, wd_sem):
    for j in range(NBUF - 1):
        _we_copy(we_hbm, we_buf, we_sem, j, j).start()
        _wd_copy(wd_hbm, wd_buf, wd_sem, j, j).start()

    def step(j, _):
        slot = jax.lax.rem(j, NBUF)

        @pl.when(j + NBUF - 1 < NF)
        def _():
            nxt = j + NBUF - 1
            nslot = jax.lax.rem(nxt, NBUF)
            _we_copy(we_hbm, we_buf, we_sem, nxt, nslot).start()
            _wd_copy(wd_hbm, wd_buf, wd_sem, nxt, nslot).start()

        _we_copy(we_hbm, we_buf, we_sem, j, slot).wait()
        f = jnp.dot(x_ref[...].astype(jnp.bfloat16),
                    we_buf[slot].astype(jnp.bfloat16),
                    preferred_element_type=jnp.float32)
        f = jnp.maximum(f + be_ref[:, pl.ds(j * BF, BF)], 0.0)
        fb = f.astype(jnp.bfloat16)

        _wd_copy(wd_hbm, wd_buf, wd_sem, j, slot).wait()
        p0 = jnp.dot(fb, wd_buf[slot, 0].astype(jnp.bfloat16),
                     preferred_element_type=jnp.float32)
        p1 = jnp.dot(fb, wd_buf[slot, 1].astype(jnp.bfloat16),
                     preferred_element_type=jnp.float32)

        @pl.when(j == 0)
        def _():
            out0_ref[...] = p0 + bd_ref[0][None]
            out1_ref[...] = p1 + bd_ref[1][None]

        @pl.when(j != 0)
        def _():
            out0_ref[...] += p0
            out1_ref[...] += p1

        return 0

    jax.lax.fori_loop(0, NF, step, 0)


@jax.jit
def kernel(x, W_enc, b_enc, W_dec, b_dec):
    x2 = x.reshape(B, L * D)
    be = b_enc.reshape(1, F)
    out0, out1 = pl.pallas_call(
        _body,
        in_specs=[
            pl.BlockSpec(memory_space=pltpu.MemorySpace.VMEM),
            pl.BlockSpec(memory_space=pltpu.MemorySpace.VMEM),
            pl.BlockSpec(memory_space=pltpu.MemorySpace.VMEM),
            pl.BlockSpec(memory_space=pl.ANY),
            pl.BlockSpec(memory_space=pl.ANY),
        ],
        out_specs=[
            pl.BlockSpec(memory_space=pltpu.MemorySpace.VMEM),
            pl.BlockSpec(memory_space=pltpu.MemorySpace.VMEM),
        ],
        out_shape=[
            jax.ShapeDtypeStruct((B, D), jnp.float32),
            jax.ShapeDtypeStruct((B, D), jnp.float32),
        ],
        scratch_shapes=[
            pltpu.VMEM((NBUF, L * D, BF), jnp.float32),
            pltpu.VMEM((NBUF, L, BF, D), jnp.float32),
            pltpu.SemaphoreType.DMA((NBUF,)),
            pltpu.SemaphoreType.DMA((NBUF,)),
        ],
    )(x2, be, b_dec, W_enc.reshape(L * D, F), W_dec)
    return jnp.stack([out0, out1], axis=1)


# speedup vs baseline: 1.0224x; 1.0224x over previous
import jax
import jax.numpy as jnp
from jax.experimental import pallas as pl
from jax.experimental.pallas import tpu as pltpu

B, L, D, F = 128, 2, 768, 32768
BF = 1024


def _body(x_ref, we_ref, be_ref, wd_ref, bd_ref, out0_ref, out1_ref):
    j = pl.program_id(0)
    f = jnp.dot(x_ref[...].astype(jnp.bfloat16), we_ref[...].astype(jnp.bfloat16),
                preferred_element_type=jnp.float32)
    f = jnp.maximum(f + be_ref[...], 0.0)
    fb = f.astype(jnp.bfloat16)
    p0 = jnp.dot(fb, wd_ref[0].astype(jnp.bfloat16),
                 preferred_element_type=jnp.float32)
    p1 = jnp.dot(fb, wd_ref[1].astype(jnp.bfloat16),
                 preferred_element_type=jnp.float32)

    @pl.when(j == 0)
    def _():
        out0_ref[...] = p0 + bd_ref[0][None]
        out1_ref[...] = p1 + bd_ref[1][None]

    @pl.when(j != 0)
    def _():
        out0_ref[...] += p0
        out1_ref[...] += p1


@jax.jit
def kernel(x, W_enc, b_enc, W_dec, b_dec):
    x2 = x.reshape(B, L * D)
    We = W_enc.reshape(L * D, F)
    be = b_enc.reshape(1, F)
    grid = (F // BF,)
    out0, out1 = pl.pallas_call(
        _body,
        grid=grid,
        in_specs=[
            pl.BlockSpec((B, L * D), lambda j: (0, 0)),
            pl.BlockSpec((L * D, BF), lambda j: (0, j)),
            pl.BlockSpec((1, BF), lambda j: (0, j)),
            pl.BlockSpec((L, BF, D), lambda j: (0, j, 0)),
            pl.BlockSpec((L, D), lambda j: (0, 0)),
        ],
        out_specs=[
            pl.BlockSpec((B, D), lambda j: (0, 0)),
            pl.BlockSpec((B, D), lambda j: (0, 0)),
        ],
        out_shape=[
            jax.ShapeDtypeStruct((B, D), jnp.float32),
            jax.ShapeDtypeStruct((B, D), jnp.float32),
        ],
        compiler_params=pltpu.CompilerParams(
            dimension_semantics=("arbitrary",),
        ),
    )(x2, We, be, W_dec, b_dec)
    return jnp.stack([out0, out1], axis=1)
